# baseline (speedup 1.0000x reference)
import functools

import jax
import jax.numpy as jnp
from jax.experimental import pallas as pl
from jax.experimental.pallas import tpu as pltpu

_BN_EPS = 1e-5


def _round_up(v, m):
    return ((v + m - 1) // m) * m


def _build_im2col(x_ref, ext_ref, im2_ref, *, W, P, pad):
    cin = x_ref.shape[1]
    ext_ref[:, :pad] = jnp.zeros((cin, pad), jnp.bfloat16)
    ext_ref[:, pad + P:] = jnp.zeros((cin, pad), jnp.bfloat16)
    ext_ref[:, pad:pad + P] = x_ref[0].astype(jnp.bfloat16)

    pix = jax.lax.broadcasted_iota(jnp.int32, (1, P), 1)
    if (W & (W - 1)) == 0:
        col = jnp.bitwise_and(pix, W - 1)
    else:
        col = jax.lax.rem(pix, W)
    not_left = col > 0
    not_right = col < (W - 1)
    col_masks = (not_left, None, not_right)

    for dy in range(3):
        for dx in range(3):
            off = (dy - 1) * W + (dx - 1)
            slab = ext_ref[:, pad + off: pad + off + P]
            mask = col_masks[dx]
            if mask is not None:
                slab = jnp.where(mask, slab, jnp.bfloat16(0))
            t = dy * 3 + dx
            im2_ref[t * cin:(t + 1) * cin, :] = slab


def _conv1(x_ref, w1_ref, ext_ref, im2_ref, *, W, P, pad):
    _build_im2col(x_ref, ext_ref, im2_ref, W=W, P=P, pad=pad)
    return jnp.dot(w1_ref[...], im2_ref[...],
                   preferred_element_type=jnp.float32)


def _stats_kernel(x_ref, w1_ref, stats_ref, ext_ref, im2_ref, *, W, P, pad):
    y = _conv1(x_ref, w1_ref, ext_ref, im2_ref, W=W, P=P, pad=pad)
    stats_ref[0, :, 0:1] = jnp.sum(y, axis=1, keepdims=True)
    stats_ref[0, :, 1:2] = jnp.sum(y * y, axis=1, keepdims=True)


def _apply_kernel(x_ref, w1_ref, scale_ref, bias_ref, w2t_ref, out_ref,
                  att_ref, ext_ref, im2_ref, *, W, P, pad):
    y = _conv1(x_ref, w1_ref, ext_ref, im2_ref, W=W, P=P, pad=pad)
    yb = jnp.maximum(y * scale_ref[...] + bias_ref[...], 0.0)
    z = jnp.dot(w2t_ref[...], yb.astype(jnp.bfloat16),
                preferred_element_type=jnp.float32)
    att = jax.nn.sigmoid(jnp.maximum(z, 0.0))
    out_ref[0] = x_ref[0] * (att + 1.0)
    att_ref[0] = att


def kernel(x_nchw, w1, gamma, beta, w2):
    N, Cin, H, W = x_nchw.shape
    Cout = w1.shape[-1]
    HW = H * W
    pad = _round_up(W + 1, 128)

    x_flat = x_nchw.astype(jnp.float32).reshape(N, Cin, HW)
    w1_flat = jnp.transpose(w1, (3, 0, 1, 2)).reshape(
        Cout, 9 * Cin).astype(jnp.bfloat16)
    w2t = jnp.transpose(w2, (1, 0)).astype(jnp.bfloat16)

    x_spec = pl.BlockSpec((1, Cin, HW), lambda n: (n, 0, 0))
    stats_spec = pl.BlockSpec((1, Cout, 2), lambda n: (n, 0, 0))
    w1_spec = pl.BlockSpec((Cout, 9 * Cin), lambda n: (0, 0))
    vec_spec = pl.BlockSpec((Cout, 1), lambda n: (0, 0))
    w2_spec = pl.BlockSpec((Cin, Cout), lambda n: (0, 0))
    scratch = [pltpu.VMEM((Cin, HW + 2 * pad), jnp.bfloat16),
               pltpu.VMEM((9 * Cin, HW), jnp.bfloat16)]

    stats = pl.pallas_call(
        functools.partial(_stats_kernel, W=W, P=HW, pad=pad),
        grid=(N,),
        in_specs=[x_spec, w1_spec],
        out_specs=stats_spec,
        out_shape=jax.ShapeDtypeStruct((N, Cout, 2), jnp.float32),
        scratch_shapes=scratch,
        compiler_params=pltpu.CompilerParams(
            dimension_semantics=("parallel",)),
    )(x_flat, w1_flat)

    n_pix = jnp.float32(N * HW)
    mean = jnp.sum(stats[:, :, 0], axis=0) / n_pix
    var = jnp.sum(stats[:, :, 1], axis=0) / n_pix - mean * mean
    inv_std = jax.lax.rsqrt(var + _BN_EPS)
    g32 = gamma.astype(jnp.float32)
    scale = (g32 * inv_std).reshape(Cout, 1)
    bias = (beta.astype(jnp.float32) - mean * g32 * inv_std).reshape(Cout, 1)

    out_flat, att_flat = pl.pallas_call(
        functools.partial(_apply_kernel, W=W, P=HW, pad=pad),
        grid=(N,),
        in_specs=[x_spec, w1_spec, vec_spec, vec_spec, w2_spec],
        out_specs=(x_spec, x_spec),
        out_shape=(jax.ShapeDtypeStruct((N, Cin, HW), jnp.float32),
                   jax.ShapeDtypeStruct((N, Cin, HW), jnp.float32)),
        scratch_shapes=scratch,
        compiler_params=pltpu.CompilerParams(
            dimension_semantics=("parallel",)),
    )(x_flat, w1_flat, scale, bias, w2t)

    return out_flat.reshape(N, Cin, H, W), att_flat.reshape(N, Cin, H, W)


# speedup vs baseline: 1.2626x; 1.2626x over previous
import functools

import jax
import jax.numpy as jnp
from jax.experimental import pallas as pl
from jax.experimental.pallas import tpu as pltpu

_BN_EPS = 1e-5


def _round_up(v, m):
    return ((v + m - 1) // m) * m


def _build_im2col(x_ref, ext_ref, im2_ref, *, W, P, pad):
    cin = x_ref.shape[1]
    ext_ref[:, :pad] = jnp.zeros((cin, pad), jnp.bfloat16)
    ext_ref[:, pad + P:] = jnp.zeros((cin, pad), jnp.bfloat16)
    ext_ref[:, pad:pad + P] = x_ref[0].astype(jnp.bfloat16)

    pix = jax.lax.broadcasted_iota(jnp.int32, (1, P), 1)
    if (W & (W - 1)) == 0:
        col = jnp.bitwise_and(pix, W - 1)
    else:
        col = jax.lax.rem(pix, W)
    not_left = col > 0
    not_right = col < (W - 1)
    col_masks = (not_left, None, not_right)

    for dy in range(3):
        for dx in range(3):
            off = (dy - 1) * W + (dx - 1)
            slab = ext_ref[:, pad + off: pad + off + P]
            mask = col_masks[dx]
            if mask is not None:
                slab = jnp.where(mask, slab, jnp.bfloat16(0))
            t = dy * 3 + dx
            im2_ref[t * cin:(t + 1) * cin, :] = slab


def _conv1(x_ref, w1_ref, ext_ref, im2_ref, *, W, P, pad):
    _build_im2col(x_ref, ext_ref, im2_ref, W=W, P=P, pad=pad)
    return jnp.dot(w1_ref[...], im2_ref[...],
                   preferred_element_type=jnp.float32)


def _stats_kernel(x_ref, w1_ref, stats_ref, ext_ref, im2_ref, *, W, P, pad):
    nb = x_ref.shape[0]
    for b in range(nb):
        y = _conv1(x_ref.at[b:b + 1], w1_ref, ext_ref, im2_ref,
                   W=W, P=P, pad=pad)
        stats_ref[b, :, 0:1] = jnp.sum(y, axis=1, keepdims=True)
        stats_ref[b, :, 1:2] = jnp.sum(y * y, axis=1, keepdims=True)


def _apply_kernel(x_ref, w1_ref, scale_ref, bias_ref, w2t_ref, out_ref,
                  att_ref, ext_ref, im2_ref, *, W, P, pad):
    nb = x_ref.shape[0]
    for b in range(nb):
        y = _conv1(x_ref.at[b:b + 1], w1_ref, ext_ref, im2_ref,
                   W=W, P=P, pad=pad)
        yb = jnp.maximum(y * scale_ref[...] + bias_ref[...], 0.0)
        z = jnp.dot(w2t_ref[...], yb.astype(jnp.bfloat16),
                    preferred_element_type=jnp.float32)
        att = jax.nn.sigmoid(jnp.maximum(z, 0.0))
        out_ref[b] = x_ref[b] * (att + 1.0)
        att_ref[b] = att


def kernel(x_nchw, w1, gamma, beta, w2):
    N, Cin, H, W = x_nchw.shape
    Cout = w1.shape[-1]
    HW = H * W
    pad = _round_up(W + 1, 128)

    x_flat = x_nchw.astype(jnp.float32).reshape(N, Cin, HW)
    w1_flat = jnp.transpose(w1, (3, 0, 1, 2)).reshape(
        Cout, 9 * Cin).astype(jnp.bfloat16)
    w2t = jnp.transpose(w2, (1, 0)).astype(jnp.bfloat16)

    B = 4 if N % 4 == 0 else 1
    x_spec = pl.BlockSpec((B, Cin, HW), lambda n: (n, 0, 0))
    stats_spec = pl.BlockSpec((B, Cout, 2), lambda n: (n, 0, 0))
    w1_spec = pl.BlockSpec((Cout, 9 * Cin), lambda n: (0, 0))
    vec_spec = pl.BlockSpec((Cout, 1), lambda n: (0, 0))
    w2_spec = pl.BlockSpec((Cin, Cout), lambda n: (0, 0))
    scratch = [pltpu.VMEM((Cin, HW + 2 * pad), jnp.bfloat16),
               pltpu.VMEM((9 * Cin, HW), jnp.bfloat16)]

    stats = pl.pallas_call(
        functools.partial(_stats_kernel, W=W, P=HW, pad=pad),
        grid=(N // B,),
        in_specs=[x_spec, w1_spec],
        out_specs=stats_spec,
        out_shape=jax.ShapeDtypeStruct((N, Cout, 2), jnp.float32),
        scratch_shapes=scratch,
        compiler_params=pltpu.CompilerParams(
            dimension_semantics=("parallel",)),
    )(x_flat, w1_flat)

    n_pix = jnp.float32(N * HW)
    mean = jnp.sum(stats[:, :, 0], axis=0) / n_pix
    var = jnp.sum(stats[:, :, 1], axis=0) / n_pix - mean * mean
    inv_std = jax.lax.rsqrt(var + _BN_EPS)
    g32 = gamma.astype(jnp.float32)
    scale = (g32 * inv_std).reshape(Cout, 1)
    bias = (beta.astype(jnp.float32) - mean * g32 * inv_std).reshape(Cout, 1)

    out_flat, att_flat = pl.pallas_call(
        functools.partial(_apply_kernel, W=W, P=HW, pad=pad),
        grid=(N // B,),
        in_specs=[x_spec, w1_spec, vec_spec, vec_spec, w2_spec],
        out_specs=(x_spec, x_spec),
        out_shape=(jax.ShapeDtypeStruct((N, Cin, HW), jnp.float32),
                   jax.ShapeDtypeStruct((N, Cin, HW), jnp.float32)),
        scratch_shapes=scratch,
        compiler_params=pltpu.CompilerParams(
            dimension_semantics=("parallel",)),
    )(x_flat, w1_flat, scale, bias, w2t)

    return out_flat.reshape(N, Cin, H, W), att_flat.reshape(N, Cin, H, W)


# speedup vs baseline: 1.2880x; 1.0201x over previous
import functools

import jax
import jax.numpy as jnp
from jax.experimental import pallas as pl
from jax.experimental.pallas import tpu as pltpu

_BN_EPS = 1e-5


def _round_up(v, m):
    return ((v + m - 1) // m) * m


def _build_im2col(x_ref, ext_ref, im2_ref, *, W, P, pad):
    cin = x_ref.shape[1]
    ext_ref[:, :pad] = jnp.zeros((cin, pad), jnp.bfloat16)
    ext_ref[:, pad + P:] = jnp.zeros((cin, pad), jnp.bfloat16)
    ext_ref[:, pad:pad + P] = x_ref[0].astype(jnp.bfloat16)

    pix = jax.lax.broadcasted_iota(jnp.int32, (1, P), 1)
    if (W & (W - 1)) == 0:
        col = jnp.bitwise_and(pix, W - 1)
    else:
        col = jax.lax.rem(pix, W)
    not_left = col > 0
    not_right = col < (W - 1)
    col_masks = (not_left, None, not_right)

    for dy in range(3):
        for dx in range(3):
            off = (dy - 1) * W + (dx - 1)
            slab = ext_ref[:, pad + off: pad + off + P]
            mask = col_masks[dx]
            if mask is not None:
                slab = jnp.where(mask, slab, jnp.bfloat16(0))
            t = dy * 3 + dx
            im2_ref[t * cin:(t + 1) * cin, :] = slab


def _conv1(x_ref, w1_ref, ext_ref, im2_ref, *, W, P, pad):
    _build_im2col(x_ref, ext_ref, im2_ref, W=W, P=P, pad=pad)
    return jnp.dot(w1_ref[...], im2_ref[...],
                   preferred_element_type=jnp.float32)


def _stats_kernel(x_ref, w1_ref, stats_ref, ext_ref, im2_ref, *, W, P, pad):
    nb = x_ref.shape[0]
    for b in range(nb):
        y = _conv1(x_ref.at[b:b + 1], w1_ref, ext_ref, im2_ref,
                   W=W, P=P, pad=pad)
        stats_ref[b, :, 0:1] = jnp.sum(y, axis=1, keepdims=True)
        stats_ref[b, :, 1:2] = jnp.sum(y * y, axis=1, keepdims=True)


def _apply_kernel(x_ref, w1_ref, scale_ref, bias_ref, w2t_ref, out_ref,
                  att_ref, ext_ref, im2_ref, *, W, P, pad):
    nb = x_ref.shape[0]
    for b in range(nb):
        y = _conv1(x_ref.at[b:b + 1], w1_ref, ext_ref, im2_ref,
                   W=W, P=P, pad=pad)
        yb = jnp.maximum(y * scale_ref[...] + bias_ref[...], 0.0)
        z = jnp.dot(w2t_ref[...], yb.astype(jnp.bfloat16),
                    preferred_element_type=jnp.float32)
        att = jax.nn.sigmoid(jnp.maximum(z, 0.0))
        out_ref[b] = x_ref[b] * (att + 1.0)
        att_ref[b] = att


def kernel(x_nchw, w1, gamma, beta, w2):
    N, Cin, H, W = x_nchw.shape
    Cout = w1.shape[-1]
    HW = H * W
    pad = _round_up(W + 1, 128)

    x_flat = x_nchw.astype(jnp.float32).reshape(N, Cin, HW)
    w1_flat = jnp.transpose(w1, (3, 0, 1, 2)).reshape(
        Cout, 9 * Cin).astype(jnp.bfloat16)
    w2t = jnp.transpose(w2, (1, 0)).astype(jnp.bfloat16)

    B = 8 if N % 8 == 0 else 1
    x_spec = pl.BlockSpec((B, Cin, HW), lambda n: (n, 0, 0))
    stats_spec = pl.BlockSpec((B, Cout, 2), lambda n: (n, 0, 0))
    w1_spec = pl.BlockSpec((Cout, 9 * Cin), lambda n: (0, 0))
    vec_spec = pl.BlockSpec((Cout, 1), lambda n: (0, 0))
    w2_spec = pl.BlockSpec((Cin, Cout), lambda n: (0, 0))
    scratch = [pltpu.VMEM((Cin, HW + 2 * pad), jnp.bfloat16),
               pltpu.VMEM((9 * Cin, HW), jnp.bfloat16)]

    stats = pl.pallas_call(
        functools.partial(_stats_kernel, W=W, P=HW, pad=pad),
        grid=(N // B,),
        in_specs=[x_spec, w1_spec],
        out_specs=stats_spec,
        out_shape=jax.ShapeDtypeStruct((N, Cout, 2), jnp.float32),
        scratch_shapes=scratch,
        compiler_params=pltpu.CompilerParams(
            dimension_semantics=("parallel",)),
    )(x_flat, w1_flat)

    n_pix = jnp.float32(N * HW)
    mean = jnp.sum(stats[:, :, 0], axis=0) / n_pix
    var = jnp.sum(stats[:, :, 1], axis=0) / n_pix - mean * mean
    inv_std = jax.lax.rsqrt(var + _BN_EPS)
    g32 = gamma.astype(jnp.float32)
    scale = (g32 * inv_std).reshape(Cout, 1)
    bias = (beta.astype(jnp.float32) - mean * g32 * inv_std).reshape(Cout, 1)

    out_flat, att_flat = pl.pallas_call(
        functools.partial(_apply_kernel, W=W, P=HW, pad=pad),
        grid=(N // B,),
        in_specs=[x_spec, w1_spec, vec_spec, vec_spec, w2_spec],
        out_specs=(x_spec, x_spec),
        out_shape=(jax.ShapeDtypeStruct((N, Cin, HW), jnp.float32),
                   jax.ShapeDtypeStruct((N, Cin, HW), jnp.float32)),
        scratch_shapes=scratch,
        compiler_params=pltpu.CompilerParams(
            dimension_semantics=("parallel",)),
    )(x_flat, w1_flat, scale, bias, w2t)

    return out_flat.reshape(N, Cin, H, W), att_flat.reshape(N, Cin, H, W)


# speedup vs baseline: 1.4380x; 1.1165x over previous
import functools

import jax
import jax.numpy as jnp
from jax.experimental import pallas as pl
from jax.experimental.pallas import tpu as pltpu

_BN_EPS = 1e-5


def _round_up(v, m):
    return ((v + m - 1) // m) * m


def _mod(v, m):
    if (m & (m - 1)) == 0:
        return jnp.bitwise_and(v, m - 1)
    return jax.lax.rem(v, m)


def _div(v, m):
    if (m & (m - 1)) == 0:
        return jnp.right_shift(v, m.bit_length() - 1)
    return jax.lax.div(v, m)


def _build_im2col(x_ref, ext_ref, im2_ref, *, H, W, P, pad):
    nb, cin, _ = x_ref.shape
    bp = nb * P
    ext_ref[:, :pad] = jnp.zeros((cin, pad), jnp.float32)
    ext_ref[:, pad + bp:] = jnp.zeros((cin, pad), jnp.float32)
    for b in range(nb):
        ext_ref[:, pad + b * P:pad + (b + 1) * P] = x_ref[b]

    pix = jax.lax.broadcasted_iota(jnp.int32, (1, bp), 1)
    col = _mod(pix, W)
    row = _mod(_div(pix, W), H)
    col_masks = (col > 0, None, col < (W - 1))
    row_masks = (row > 0, None, row < (H - 1))

    for dy in range(3):
        for dx in range(3):
            off = (dy - 1) * W + (dx - 1)
            slab = ext_ref[:, pad + off: pad + off + bp]
            mask = row_masks[dy]
            if col_masks[dx] is not None:
                mask = col_masks[dx] if mask is None else jnp.logical_and(
                    mask, col_masks[dx])
            if mask is not None:
                slab = jnp.where(mask, slab, 0.0)
            t = dy * 3 + dx
            im2_ref[t * cin:(t + 1) * cin, :] = slab.astype(jnp.bfloat16)


def _conv1(x_ref, w1_ref, ext_ref, im2_ref, *, H, W, P, pad):
    _build_im2col(x_ref, ext_ref, im2_ref, H=H, W=W, P=P, pad=pad)
    return jnp.dot(w1_ref[...], im2_ref[...],
                   preferred_element_type=jnp.float32)


def _stats_kernel(x_ref, w1_ref, stats_ref, ext_ref, im2_ref,
                  *, H, W, P, pad):
    nb = x_ref.shape[0]
    y = _conv1(x_ref, w1_ref, ext_ref, im2_ref, H=H, W=W, P=P, pad=pad)
    for b in range(nb):
        ys = y[:, b * P:(b + 1) * P]
        stats_ref[b, :, 0:1] = jnp.sum(ys, axis=1, keepdims=True)
        stats_ref[b, :, 1:2] = jnp.sum(ys * ys, axis=1, keepdims=True)


def _apply_kernel(x_ref, w1_ref, scale_ref, bias_ref, w2t_ref, out_ref,
                  att_ref, ext_ref, im2_ref, *, H, W, P, pad):
    nb = x_ref.shape[0]
    y = _conv1(x_ref, w1_ref, ext_ref, im2_ref, H=H, W=W, P=P, pad=pad)
    yb = jnp.maximum(y * scale_ref[...] + bias_ref[...], 0.0)
    z = jnp.dot(w2t_ref[...], yb.astype(jnp.bfloat16),
                preferred_element_type=jnp.float32)
    att = jax.nn.sigmoid(jnp.maximum(z, 0.0))
    for b in range(nb):
        a = att[:, b * P:(b + 1) * P]
        out_ref[b] = x_ref[b] * (a + 1.0)
        att_ref[b] = a


def kernel(x_nchw, w1, gamma, beta, w2):
    N, Cin, H, W = x_nchw.shape
    Cout = w1.shape[-1]
    HW = H * W
    pad = _round_up(W + 1, 128)
    B = 8 if N % 8 == 0 else 1

    x_flat = x_nchw.astype(jnp.float32).reshape(N, Cin, HW)
    w1_flat = jnp.transpose(w1, (3, 0, 1, 2)).reshape(
        Cout, 9 * Cin).astype(jnp.bfloat16)
    w2t = jnp.transpose(w2, (1, 0)).astype(jnp.bfloat16)

    x_spec = pl.BlockSpec((B, Cin, HW), lambda n: (n, 0, 0))
    stats_spec = pl.BlockSpec((B, Cout, 2), lambda n: (n, 0, 0))
    w1_spec = pl.BlockSpec((Cout, 9 * Cin), lambda n: (0, 0))
    vec_spec = pl.BlockSpec((Cout, 1), lambda n: (0, 0))
    w2_spec = pl.BlockSpec((Cin, Cout), lambda n: (0, 0))
    scratch = [pltpu.VMEM((Cin, B * HW + 2 * pad), jnp.float32),
               pltpu.VMEM((9 * Cin, B * HW), jnp.bfloat16)]

    stats = pl.pallas_call(
        functools.partial(_stats_kernel, H=H, W=W, P=HW, pad=pad),
        grid=(N // B,),
        in_specs=[x_spec, w1_spec],
        out_specs=stats_spec,
        out_shape=jax.ShapeDtypeStruct((N, Cout, 2), jnp.float32),
        scratch_shapes=scratch,
        compiler_params=pltpu.CompilerParams(
            dimension_semantics=("parallel",)),
    )(x_flat, w1_flat)

    n_pix = jnp.float32(N * HW)
    mean = jnp.sum(stats[:, :, 0], axis=0) / n_pix
    var = jnp.sum(stats[:, :, 1], axis=0) / n_pix - mean * mean
    inv_std = jax.lax.rsqrt(var + _BN_EPS)
    g32 = gamma.astype(jnp.float32)
    scale = (g32 * inv_std).reshape(Cout, 1)
    bias = (beta.astype(jnp.float32) - mean * g32 * inv_std).reshape(Cout, 1)

    out_flat, att_flat = pl.pallas_call(
        functools.partial(_apply_kernel, H=H, W=W, P=HW, pad=pad),
        grid=(N // B,),
        in_specs=[x_spec, w1_spec, vec_spec, vec_spec, w2_spec],
        out_specs=(x_spec, x_spec),
        out_shape=(jax.ShapeDtypeStruct((N, Cin, HW), jnp.float32),
                   jax.ShapeDtypeStruct((N, Cin, HW), jnp.float32)),
        scratch_shapes=scratch,
        compiler_params=pltpu.CompilerParams(
            dimension_semantics=("parallel",)),
    )(x_flat, w1_flat, scale, bias, w2t)

    return out_flat.reshape(N, Cin, H, W), att_flat.reshape(N, Cin, H, W)
